# Initial kernel scaffold; baseline (speedup 1.0000x reference)
import jax
import jax.numpy as jnp
from jax.experimental import pallas as pl


def kernel(x, edge_index, W1_l, W1_r, b1, W2_l, W2_r, b2):
    raise NotImplementedError("write your pallas kernel here")


# speedup vs baseline: 5.2585x; 5.2585x over previous
import functools

import jax
import jax.numpy as jnp
from jax import lax
from jax.experimental import pallas as pl
from jax.experimental.pallas import tpu as pltpu
from jax.experimental.pallas import tpu_sc as plsc

NC = 2
NS = 16
NW = NC * NS
LANES = 16

CHUNK = 128
N_NODES = 10000
D_FEAT = 128

E_EDGES = 320000
CHUNKS_PER_W = -(-E_EDGES // (NW * CHUNK))
EDGES_PER_W = CHUNKS_PER_W * CHUNK
E_PAD = NW * EDGES_PER_W
DUMMY_ROW = N_NODES
N_PAD = 10240
ACC_ROWS = N_PAD
RPT = ACC_ROWS // NS


def _make_sc_segsum(with_count: bool):
  mesh = plsc.VectorSubcoreMesh(core_axis_name="c", subcore_axis_name="s")

  out_type = [jax.ShapeDtypeStruct((NC, ACC_ROWS, D_FEAT), jnp.float32)]
  scratch = [
      pltpu.VMEM_SHARED((ACC_ROWS, D_FEAT), jnp.float32),
      pltpu.VMEM((CHUNK,), jnp.int32),
      pltpu.VMEM((CHUNK,), jnp.int32),
      pltpu.VMEM((CHUNK, D_FEAT), jnp.float32),
      pltpu.SemaphoreType.DMA,
  ]
  if with_count:
    out_type.append(jax.ShapeDtypeStruct((NC, ACC_ROWS), jnp.float32))
    scratch.append(pltpu.VMEM_SHARED((ACC_ROWS,), jnp.float32))
    scratch.append(pltpu.VMEM((CHUNK,), jnp.float32))

  def body(table_hbm, src_hbm, dst_hbm, z2_hbm, z1_hbm, *rest):
    if with_count:
      agg_out, cnt_out, acc_sh, src_v, dst_v, rows_v, sem, cnt_sh, ones_v = rest
    else:
      agg_out, acc_sh, src_v, dst_v, rows_v, sem = rest
    c = lax.axis_index("c")
    s = lax.axis_index("s")
    wid = s * NC + c

    pltpu.sync_copy(z2_hbm.at[pl.ds(s * RPT, RPT)], acc_sh.at[pl.ds(s * RPT, RPT)])
    if with_count:
      pltpu.sync_copy(z1_hbm.at[pl.ds(s * RPT, RPT)], cnt_sh.at[pl.ds(s * RPT, RPT)])
      ones16 = jnp.ones((LANES,), jnp.float32)
      for k in range(CHUNK // LANES):
        ones_v[pl.ds(k * LANES, LANES)] = ones16
    plsc.subcore_barrier()

    base = wid * EDGES_PER_W

    def chunk_body(j, carry):
      off = base + j * CHUNK
      pltpu.sync_copy(src_hbm.at[pl.ds(off, CHUNK)], src_v)
      pltpu.sync_copy(dst_hbm.at[pl.ds(off, CHUNK)], dst_v)
      pltpu.async_copy(table_hbm.at[src_v], rows_v, sem).wait()
      pltpu.sync_copy(rows_v, acc_sh.at[dst_v], add=True)
      if with_count:
        pltpu.sync_copy(ones_v, cnt_sh.at[dst_v], add=True)
      return carry

    lax.fori_loop(0, CHUNKS_PER_W, chunk_body, 0)
    plsc.subcore_barrier()

    pltpu.sync_copy(acc_sh.at[pl.ds(s * RPT, RPT)],
                    agg_out.at[c, pl.ds(s * RPT, RPT)])
    if with_count:
      pltpu.sync_copy(cnt_sh.at[pl.ds(s * RPT, RPT)],
                      cnt_out.at[c, pl.ds(s * RPT, RPT)])

  return pl.kernel(body, out_type=tuple(out_type), mesh=mesh,
                   scratch_types=scratch)


_sc_segsum_count = _make_sc_segsum(True)
_sc_segsum = _make_sc_segsum(False)


def _layer1_body(agg_ref, cnt_ref, x_ref, wl_ref, wr_ref, b_ref, w2l_ref,
                 h_ref, p_ref):
  cnt = jnp.maximum(jnp.sum(cnt_ref[...], axis=0), 1.0)
  agg = agg_ref[0] + agg_ref[1]
  mean = agg / cnt[:, None]
  h = jnp.tanh(
      jnp.dot(mean, wl_ref[...], preferred_element_type=jnp.float32)
      + jnp.dot(x_ref[...], wr_ref[...], preferred_element_type=jnp.float32)
      + b_ref[...])
  h_ref[...] = h
  p_ref[...] = jnp.dot(h, w2l_ref[...], preferred_element_type=jnp.float32)


def _layer2_body(agg_ref, cnt_ref, h_ref, wr_ref, b_ref, out_ref):
  cnt = jnp.maximum(jnp.sum(cnt_ref[...], axis=0), 1.0)
  mean_l = (agg_ref[0] + agg_ref[1]) / cnt[:, None]
  out_ref[...] = jnp.tanh(
      mean_l
      + jnp.dot(h_ref[...], wr_ref[...], preferred_element_type=jnp.float32)
      + b_ref[...])


_BLK = 1024


def kernel(x, edge_index, W1_l, W1_r, b1, W2_l, W2_r, b2):
  src = edge_index[0].astype(jnp.int32)
  dst = edge_index[1].astype(jnp.int32)
  pad = E_PAD - src.shape[0]
  src_p = jnp.concatenate([src, jnp.zeros((pad,), jnp.int32)])
  dst_p = jnp.concatenate([dst, jnp.full((pad,), DUMMY_ROW, jnp.int32)])
  z2 = jnp.zeros((ACC_ROWS, D_FEAT), jnp.float32)
  z1 = jnp.zeros((ACC_ROWS,), jnp.float32)

  n = x.shape[0]
  x_pad = jnp.pad(x, ((0, N_PAD - n), (0, 0)))

  agg1, cnt_part = _sc_segsum_count(x_pad, src_p, dst_p, z2, z1)

  grid = N_PAD // _BLK
  h, p = pl.pallas_call(
      _layer1_body,
      grid=(grid,),
      in_specs=[
          pl.BlockSpec((NC, _BLK, D_FEAT), lambda i: (0, i, 0)),
          pl.BlockSpec((NC, _BLK), lambda i: (0, i)),
          pl.BlockSpec((_BLK, D_FEAT), lambda i: (i, 0)),
          pl.BlockSpec((D_FEAT, 256), lambda i: (0, 0)),
          pl.BlockSpec((D_FEAT, 256), lambda i: (0, 0)),
          pl.BlockSpec((1, 256), lambda i: (0, 0)),
          pl.BlockSpec((256, D_FEAT), lambda i: (0, 0)),
      ],
      out_specs=[
          pl.BlockSpec((_BLK, 256), lambda i: (i, 0)),
          pl.BlockSpec((_BLK, D_FEAT), lambda i: (i, 0)),
      ],
      out_shape=[
          jax.ShapeDtypeStruct((N_PAD, 256), jnp.float32),
          jax.ShapeDtypeStruct((N_PAD, D_FEAT), jnp.float32),
      ],
  )(agg1, cnt_part, x_pad, W1_l, W1_r, b1.reshape(1, 256), W2_l)

  (agg2,) = _sc_segsum(p, src_p, dst_p, z2, z1)

  out = pl.pallas_call(
      _layer2_body,
      grid=(grid,),
      in_specs=[
          pl.BlockSpec((NC, _BLK, D_FEAT), lambda i: (0, i, 0)),
          pl.BlockSpec((NC, _BLK), lambda i: (0, i)),
          pl.BlockSpec((_BLK, 256), lambda i: (i, 0)),
          pl.BlockSpec((256, D_FEAT), lambda i: (0, 0)),
          pl.BlockSpec((1, D_FEAT), lambda i: (0, 0)),
      ],
      out_specs=pl.BlockSpec((_BLK, D_FEAT), lambda i: (i, 0)),
      out_shape=jax.ShapeDtypeStruct((N_PAD, D_FEAT), jnp.float32),
  )(agg2, cnt_part, h, W2_r, b2.reshape(1, D_FEAT))

  return out[:n]
